# baseline (device time: 35096 ns/iter reference)
import jax
import jax.numpy as jnp
from jax import lax
from jax.experimental import pallas as pl
from jax.experimental.pallas import tpu as pltpu

N_DEV = 4
B, SQ, SKV, DH = 2, 256, 256, 64
HQ_LOCAL = 4
D_MODEL = 512
D_QKV = HQ_LOCAL * DH
BLK = 64


def kernel(x, Wq, K_ext, V_ext, Wo):
    x2 = x.reshape(B * SQ, D_MODEL)
    K2 = jnp.transpose(K_ext, (0, 2, 1, 3)).reshape(B * HQ_LOCAL, SKV, DH)
    V2 = jnp.transpose(V_ext, (0, 2, 1, 3)).reshape(B * HQ_LOCAL, SKV, DH)

    def body(x_ref, wq_ref, k_ref, v_ref, wo_ref, out_ref,
             comm_ref, send_sems, recv_sems):
        my_pos = lax.axis_index("i")
        left = (my_pos + N_DEV - 1) % N_DEV
        right = (my_pos + 1) % N_DEV

        barrier_sem = pltpu.get_barrier_semaphore()
        for nbr in [left, right]:
            pl.semaphore_signal(
                barrier_sem, inc=1,
                device_id=(nbr,), device_id_type=pl.DeviceIdType.MESH,
            )
        pl.semaphore_wait(barrier_sem, 2)

        wq_loc = wq_ref[:, pl.ds(my_pos * D_QKV, D_QKV)].astype(jnp.bfloat16)
        q = jnp.dot(x_ref[...].astype(jnp.bfloat16), wq_loc,
                    preferred_element_type=jnp.float32)

        qb = lax.broadcasted_iota(jnp.int32, (SQ, SKV), 0) // BLK
        kb = lax.broadcasted_iota(jnp.int32, (SQ, SKV), 1) // BLK
        mask = qb == kb

        for b in range(B):
            acc = jnp.zeros((SQ, D_MODEL), jnp.float32)
            for h in range(HQ_LOCAL):
                q_bh = q[b * SQ:(b + 1) * SQ,
                         h * DH:(h + 1) * DH].astype(jnp.bfloat16)
                k_bh = k_ref[b * HQ_LOCAL + h].astype(jnp.bfloat16)
                v_bh = v_ref[b * HQ_LOCAL + h].astype(jnp.bfloat16)
                s = lax.dot_general(
                    q_bh, k_bh, (((1,), (1,)), ((), ())),
                    preferred_element_type=jnp.float32) * 0.125
                s = jnp.where(mask, s, -1e9)
                m = jnp.max(s, axis=1, keepdims=True)
                w = jnp.exp(s - m)
                w = w / jnp.sum(w, axis=1, keepdims=True)
                ctx = jnp.dot(w.astype(jnp.bfloat16), v_bh,
                              preferred_element_type=jnp.float32)
                wo_h = wo_ref[pl.ds((my_pos * HQ_LOCAL + h) * DH, DH),
                              :].astype(jnp.bfloat16)
                acc = acc + jnp.dot(ctx.astype(jnp.bfloat16), wo_h,
                                    preferred_element_type=jnp.float32)
            out_ref[b * SQ:(b + 1) * SQ, :] = acc
            comm_ref[0, b * SQ:(b + 1) * SQ, :] = acc.astype(jnp.bfloat16)

        for hop in range(N_DEV - 1):
            rdma = pltpu.make_async_remote_copy(
                src_ref=comm_ref.at[hop],
                dst_ref=comm_ref.at[hop + 1],
                send_sem=send_sems.at[hop],
                recv_sem=recv_sems.at[hop],
                device_id=(right,),
                device_id_type=pl.DeviceIdType.MESH,
            )
            rdma.start()
            rdma.wait()
            out_ref[...] += comm_ref[hop + 1].astype(jnp.float32)

    out2 = pl.pallas_call(
        body,
        out_shape=jax.ShapeDtypeStruct((B * SQ, D_MODEL), jnp.float32),
        in_specs=[pl.BlockSpec(memory_space=pltpu.VMEM)] * 5,
        out_specs=pl.BlockSpec(memory_space=pltpu.VMEM),
        scratch_shapes=[
            pltpu.VMEM((N_DEV, B * SQ, D_MODEL), jnp.bfloat16),
            pltpu.SemaphoreType.DMA((N_DEV - 1,)),
            pltpu.SemaphoreType.DMA((N_DEV - 1,)),
        ],
        compiler_params=pltpu.CompilerParams(collective_id=0),
    )(x2, Wq, K2, V2, Wo)

    return out2.reshape(B, SQ, D_MODEL)


# device time: 22712 ns/iter; 1.5453x vs baseline; 1.5453x over previous
import jax
import jax.numpy as jnp
from jax import lax
from jax.experimental import pallas as pl
from jax.experimental.pallas import tpu as pltpu

N_DEV = 4
B, SQ, SKV, DH = 2, 256, 256, 64
HQ_LOCAL = 4
D_MODEL = 512
D_QKV = HQ_LOCAL * DH
BLK = 64


def kernel(x, Wq, K_ext, V_ext, Wo):
    x2 = x.reshape(B * SQ, D_MODEL)
    K2 = jnp.transpose(K_ext, (0, 2, 1, 3)).reshape(B * HQ_LOCAL, SKV, DH)
    V2 = jnp.transpose(V_ext, (0, 2, 1, 3)).reshape(B * HQ_LOCAL, SKV, DH)

    def body(x_ref, wq_ref, k_ref, v_ref, wo_ref, out_ref,
             send_buf, recv_buf, send_sems, recv_sems):
        my_pos = lax.axis_index("i")
        left = (my_pos + N_DEV - 1) % N_DEV
        right = (my_pos + 1) % N_DEV
        partners = [jnp.bitwise_xor(my_pos, 1), 3 - my_pos]

        barrier_sem = pltpu.get_barrier_semaphore()
        for nbr in [left, right]:
            pl.semaphore_signal(
                barrier_sem, inc=1,
                device_id=(nbr,), device_id_type=pl.DeviceIdType.MESH,
            )
        pl.semaphore_wait(barrier_sem, 2)

        def exchange(rnd, b):
            return pltpu.make_async_remote_copy(
                src_ref=send_buf.at[rnd, b],
                dst_ref=recv_buf.at[rnd, b],
                send_sem=send_sems.at[rnd, b],
                recv_sem=recv_sems.at[rnd, b],
                device_id=(partners[rnd],),
                device_id_type=pl.DeviceIdType.MESH,
            )

        wq_loc = wq_ref[:, pl.ds(my_pos * D_QKV, D_QKV)].astype(jnp.bfloat16)
        q = jnp.dot(x_ref[...].astype(jnp.bfloat16), wq_loc,
                    preferred_element_type=jnp.float32)

        qb = lax.broadcasted_iota(jnp.int32, (SQ, SKV), 0) // BLK
        kb = lax.broadcasted_iota(jnp.int32, (SQ, SKV), 1) // BLK
        mask = qb == kb

        def attn_partial(b):
            acc = jnp.zeros((SQ, D_MODEL), jnp.float32)
            for h in range(HQ_LOCAL):
                q_bh = q[b * SQ:(b + 1) * SQ,
                         h * DH:(h + 1) * DH].astype(jnp.bfloat16)
                k_bh = k_ref[b * HQ_LOCAL + h].astype(jnp.bfloat16)
                v_bh = v_ref[b * HQ_LOCAL + h].astype(jnp.bfloat16)
                s = lax.dot_general(
                    q_bh, k_bh, (((1,), (1,)), ((), ())),
                    preferred_element_type=jnp.float32) * 0.125
                s = jnp.where(mask, s, -1e9)
                m = jnp.max(s, axis=1, keepdims=True)
                w = jnp.exp(s - m)
                w = w / jnp.sum(w, axis=1, keepdims=True)
                ctx = jnp.dot(w.astype(jnp.bfloat16), v_bh,
                              preferred_element_type=jnp.float32)
                wo_h = wo_ref[pl.ds((my_pos * HQ_LOCAL + h) * DH, DH),
                              :].astype(jnp.bfloat16)
                acc = acc + jnp.dot(ctx.astype(jnp.bfloat16), wo_h,
                                    preferred_element_type=jnp.float32)
            return acc

        acc = [None, None]
        r0 = [None, None]
        for b in range(B):
            acc[b] = attn_partial(b)
            send_buf[0, b] = acc[b].astype(jnp.bfloat16)
            r0[b] = exchange(0, b)
            r0[b].start()

        acc2 = [None, None]
        r1 = [None, None]
        for b in range(B):
            r0[b].wait_recv()
            acc2[b] = acc[b] + recv_buf[0, b].astype(jnp.float32)
            send_buf[1, b] = acc2[b].astype(jnp.bfloat16)
            r1[b] = exchange(1, b)
            r1[b].start()

        for b in range(B):
            r1[b].wait_recv()
            out_ref[b * SQ:(b + 1) * SQ, :] = (
                acc2[b] + recv_buf[1, b].astype(jnp.float32))

        for b in range(B):
            r0[b].wait_send()
            r1[b].wait_send()

    out2 = pl.pallas_call(
        body,
        out_shape=jax.ShapeDtypeStruct((B * SQ, D_MODEL), jnp.float32),
        in_specs=[pl.BlockSpec(memory_space=pltpu.VMEM)] * 5,
        out_specs=pl.BlockSpec(memory_space=pltpu.VMEM),
        scratch_shapes=[
            pltpu.VMEM((2, B, SQ, D_MODEL), jnp.bfloat16),
            pltpu.VMEM((2, B, SQ, D_MODEL), jnp.bfloat16),
            pltpu.SemaphoreType.DMA((2, B)),
            pltpu.SemaphoreType.DMA((2, B)),
        ],
        compiler_params=pltpu.CompilerParams(collective_id=0),
    )(x2, Wq, K2, V2, Wo)

    return out2.reshape(B, SQ, D_MODEL)


# device time: 22446 ns/iter; 1.5636x vs baseline; 1.0119x over previous
import jax
import jax.numpy as jnp
from jax import lax
from jax.experimental import pallas as pl
from jax.experimental.pallas import tpu as pltpu

N_DEV = 4
B, SQ, SKV, DH = 2, 256, 256, 64
HQ_LOCAL = 4
D_MODEL = 512
D_QKV = HQ_LOCAL * DH
BLK = 64


def kernel(x, Wq, K_ext, V_ext, Wo):
    x2 = x.reshape(B * SQ, D_MODEL)
    K3 = K_ext.reshape(B, SKV, HQ_LOCAL * DH)
    V3 = V_ext.reshape(B, SKV, HQ_LOCAL * DH)

    def body(x_ref, wq_ref, k_ref, v_ref, wo_ref, out_ref,
             send_buf, recv_buf, send_sems, recv_sems):
        my_pos = lax.axis_index("i")
        left = (my_pos + N_DEV - 1) % N_DEV
        right = (my_pos + 1) % N_DEV
        partners = [jnp.bitwise_xor(my_pos, 1), 3 - my_pos]

        barrier_sem = pltpu.get_barrier_semaphore()
        for nbr in [left, right]:
            pl.semaphore_signal(
                barrier_sem, inc=1,
                device_id=(nbr,), device_id_type=pl.DeviceIdType.MESH,
            )
        pl.semaphore_wait(barrier_sem, 2)

        def exchange(rnd, b):
            return pltpu.make_async_remote_copy(
                src_ref=send_buf.at[rnd, b],
                dst_ref=recv_buf.at[rnd, b],
                send_sem=send_sems.at[rnd, b],
                recv_sem=recv_sems.at[rnd, b],
                device_id=(partners[rnd],),
                device_id_type=pl.DeviceIdType.MESH,
            )

        wq_loc = wq_ref[:, pl.ds(my_pos * D_QKV, D_QKV)].astype(jnp.bfloat16)
        q = jnp.dot(x_ref[...].astype(jnp.bfloat16), wq_loc,
                    preferred_element_type=jnp.float32
                    ).astype(jnp.bfloat16)
        wo_loc = wo_ref[pl.ds(my_pos * D_QKV, D_QKV), :].astype(jnp.bfloat16)

        qb = lax.broadcasted_iota(jnp.int32, (SQ, SKV), 0) // BLK
        kb = lax.broadcasted_iota(jnp.int32, (SQ, SKV), 1) // BLK
        bias = jnp.where(qb == kb, 0.0, -30.0).astype(jnp.float32)

        def attn_partial(b):
            kv_b = k_ref[b].astype(jnp.bfloat16)
            vv_b = v_ref[b].astype(jnp.bfloat16)
            ctxs = []
            for h in range(HQ_LOCAL):
                q_bh = q[b * SQ:(b + 1) * SQ, h * DH:(h + 1) * DH]
                k_bh = kv_b[:, h * DH:(h + 1) * DH]
                v_bh = vv_b[:, h * DH:(h + 1) * DH]
                s = lax.dot_general(
                    q_bh, k_bh, (((1,), (1,)), ((), ())),
                    preferred_element_type=jnp.float32) * 0.125 + bias
                w = jnp.exp(s)
                denom = jnp.sum(w, axis=1, keepdims=True)
                ctx = jnp.dot(w.astype(jnp.bfloat16), v_bh,
                              preferred_element_type=jnp.float32)
                ctxs.append((ctx / denom).astype(jnp.bfloat16))
            ctx_all = jnp.concatenate(ctxs, axis=1)
            return jnp.dot(ctx_all, wo_loc,
                           preferred_element_type=jnp.float32)

        acc = [None, None]
        r0 = [None, None]
        for b in range(B):
            acc[b] = attn_partial(b)
            send_buf[0, b] = acc[b].astype(jnp.bfloat16)
            r0[b] = exchange(0, b)
            r0[b].start()

        acc2 = [None, None]
        r1 = [None, None]
        for b in range(B):
            r0[b].wait_recv()
            acc2[b] = acc[b] + recv_buf[0, b].astype(jnp.float32)
            send_buf[1, b] = acc2[b].astype(jnp.bfloat16)
            r1[b] = exchange(1, b)
            r1[b].start()

        for b in range(B):
            r1[b].wait_recv()
            out_ref[b * SQ:(b + 1) * SQ, :] = (
                acc2[b] + recv_buf[1, b].astype(jnp.float32))

        for b in range(B):
            r0[b].wait_send()
            r1[b].wait_send()

    out2 = pl.pallas_call(
        body,
        out_shape=jax.ShapeDtypeStruct((B * SQ, D_MODEL), jnp.float32),
        in_specs=[pl.BlockSpec(memory_space=pltpu.VMEM)] * 5,
        out_specs=pl.BlockSpec(memory_space=pltpu.VMEM),
        scratch_shapes=[
            pltpu.VMEM((2, B, SQ, D_MODEL), jnp.bfloat16),
            pltpu.VMEM((2, B, SQ, D_MODEL), jnp.bfloat16),
            pltpu.SemaphoreType.DMA((2, B)),
            pltpu.SemaphoreType.DMA((2, B)),
        ],
        compiler_params=pltpu.CompilerParams(collective_id=0),
    )(x2, Wq, K3, V3, Wo)

    return out2.reshape(B, SQ, D_MODEL)


# device time: 22306 ns/iter; 1.5734x vs baseline; 1.0063x over previous
import jax
import jax.numpy as jnp
from jax import lax
from jax.experimental import pallas as pl
from jax.experimental.pallas import tpu as pltpu

N_DEV = 4
B, SQ, SKV, DH = 2, 256, 256, 64
HQ_LOCAL = 4
D_MODEL = 512
D_QKV = HQ_LOCAL * DH
BLK = 64


def kernel(x, Wq, K_ext, V_ext, Wo):
    def body(x_ref, wq_ref, k_ref, v_ref, wo_ref, out_ref,
             send_buf, recv_buf, send_sems, recv_sems):
        my_pos = lax.axis_index("i")
        left = (my_pos + N_DEV - 1) % N_DEV
        right = (my_pos + 1) % N_DEV
        partners = [jnp.bitwise_xor(my_pos, 1), 3 - my_pos]

        barrier_sem = pltpu.get_barrier_semaphore()
        for nbr in [left, right]:
            pl.semaphore_signal(
                barrier_sem, inc=1,
                device_id=(nbr,), device_id_type=pl.DeviceIdType.MESH,
            )
        pl.semaphore_wait(barrier_sem, 2)

        def exchange(rnd, b):
            return pltpu.make_async_remote_copy(
                src_ref=send_buf.at[rnd, b],
                dst_ref=recv_buf.at[rnd, b],
                send_sem=send_sems.at[rnd, b],
                recv_sem=recv_sems.at[rnd, b],
                device_id=(partners[rnd],),
                device_id_type=pl.DeviceIdType.MESH,
            )

        wq_loc = wq_ref[:, pl.ds(my_pos * D_QKV, D_QKV)].astype(jnp.bfloat16)
        wo_loc = wo_ref[pl.ds(my_pos * D_QKV, D_QKV), :].astype(jnp.bfloat16)

        qb = lax.broadcasted_iota(jnp.int32, (SQ, SKV), 0) // BLK
        kb = lax.broadcasted_iota(jnp.int32, (SQ, SKV), 1) // BLK
        bias = jnp.where(qb == kb, 0.0, -30.0).astype(jnp.float32)

        def attn_partial(b):
            q_b = jnp.dot(x_ref[b].astype(jnp.bfloat16), wq_loc,
                          preferred_element_type=jnp.float32
                          ).astype(jnp.bfloat16)
            ctxs = []
            for h in range(HQ_LOCAL):
                q_bh = q_b[:, h * DH:(h + 1) * DH]
                k_bh = k_ref[b, :, h, :].astype(jnp.bfloat16)
                v_bh = v_ref[b, :, h, :].astype(jnp.bfloat16)
                s = lax.dot_general(
                    q_bh, k_bh, (((1,), (1,)), ((), ())),
                    preferred_element_type=jnp.float32) * 0.125 + bias
                w = jnp.exp(s)
                denom = jnp.sum(w, axis=1, keepdims=True)
                ctx = jnp.dot(w.astype(jnp.bfloat16), v_bh,
                              preferred_element_type=jnp.float32)
                ctxs.append((ctx / denom).astype(jnp.bfloat16))
            ctx_all = jnp.concatenate(ctxs, axis=1)
            return jnp.dot(ctx_all, wo_loc,
                           preferred_element_type=jnp.float32)

        acc = [None, None]
        r0 = [None, None]
        for b in range(B):
            acc[b] = attn_partial(b)
            send_buf[0, b] = acc[b].astype(jnp.bfloat16)
            r0[b] = exchange(0, b)
            r0[b].start()

        acc2 = [None, None]
        r1 = [None, None]
        for b in range(B):
            r0[b].wait_recv()
            acc2[b] = acc[b] + recv_buf[0, b].astype(jnp.float32)
            send_buf[1, b] = acc2[b].astype(jnp.bfloat16)
            r1[b] = exchange(1, b)
            r1[b].start()

        for b in range(B):
            r1[b].wait_recv()
            out_ref[b] = acc2[b] + recv_buf[1, b].astype(jnp.float32)

        for b in range(B):
            r0[b].wait_send()
            r1[b].wait_send()

    return pl.pallas_call(
        body,
        out_shape=jax.ShapeDtypeStruct((B, SQ, D_MODEL), jnp.float32),
        in_specs=[pl.BlockSpec(memory_space=pltpu.VMEM)] * 5,
        out_specs=pl.BlockSpec(memory_space=pltpu.VMEM),
        scratch_shapes=[
            pltpu.VMEM((2, B, SQ, D_MODEL), jnp.bfloat16),
            pltpu.VMEM((2, B, SQ, D_MODEL), jnp.bfloat16),
            pltpu.SemaphoreType.DMA((2, B)),
            pltpu.SemaphoreType.DMA((2, B)),
        ],
        compiler_params=pltpu.CompilerParams(collective_id=0),
    )(x, Wq, K_ext, V_ext, Wo)


# device time: 14813 ns/iter; 2.3693x vs baseline; 1.5058x over previous
import jax
import jax.numpy as jnp
from jax import lax
from jax.experimental import pallas as pl
from jax.experimental.pallas import tpu as pltpu

N_DEV = 4
B, SQ, SKV, DH = 2, 256, 256, 64
HQ_LOCAL = 4
D_MODEL = 512
D_QKV = HQ_LOCAL * DH
BLK = 64
NC_PER_B = 4
NC = NC_PER_B * B
CW = D_MODEL // NC_PER_B


def kernel(x, Wq, K_ext, V_ext, Wo):
    Kt = jnp.transpose(K_ext, (0, 2, 3, 1))
    Vt = jnp.transpose(V_ext, (0, 2, 3, 1))
    my_pos_out = lax.axis_index("i")
    wq_loc_out = lax.dynamic_slice(Wq, (0, my_pos_out * D_QKV),
                                   (D_MODEL, D_QKV))
    wo_loc_out = lax.dynamic_slice(Wo, (my_pos_out * D_QKV, 0),
                                   (D_QKV, D_MODEL))

    def body(x_ref, wq_ref, k_ref, v_ref, wo_ref, out_ref,
             send_buf, recv_buf, send_sems, recv_sems):
        my_pos = lax.axis_index("i")
        left = (my_pos + N_DEV - 1) % N_DEV
        right = (my_pos + 1) % N_DEV
        partners = [jnp.bitwise_xor(my_pos, 1), 3 - my_pos]

        barrier_sem = pltpu.get_barrier_semaphore()
        for nbr in [left, right]:
            pl.semaphore_signal(
                barrier_sem, inc=1,
                device_id=(nbr,), device_id_type=pl.DeviceIdType.MESH,
            )

        def exchange(rnd, c):
            return pltpu.make_async_remote_copy(
                src_ref=send_buf.at[rnd, c],
                dst_ref=recv_buf.at[rnd, c],
                send_sem=send_sems.at[rnd, c],
                recv_sem=recv_sems.at[rnd, c],
                device_id=(partners[(c // NC_PER_B) ^ rnd],),
                device_id_type=pl.DeviceIdType.MESH,
            )

        wq_loc = wq_ref[...].astype(jnp.bfloat16)
        wo_loc = wo_ref[...].astype(jnp.bfloat16)

        qb = lax.broadcasted_iota(jnp.int32, (SQ, SKV), 0) // BLK
        kb = lax.broadcasted_iota(jnp.int32, (SQ, SKV), 1) // BLK
        bias = jnp.where(qb == kb, 0.0, -30.0).astype(jnp.float32)

        def attn_partial(b):
            q_b = jnp.dot(x_ref[b].astype(jnp.bfloat16), wq_loc,
                          preferred_element_type=jnp.float32
                          ).astype(jnp.bfloat16)
            ctxs = []
            for h in range(HQ_LOCAL):
                q_bh = q_b[:, h * DH:(h + 1) * DH]
                kt_bh = k_ref[b, h].astype(jnp.bfloat16)
                vt_bh = v_ref[b, h].astype(jnp.bfloat16)
                s = jnp.dot(q_bh, kt_bh,
                            preferred_element_type=jnp.float32) * 0.125 + bias
                w = jnp.exp(s)
                denom = jnp.sum(w, axis=1, keepdims=True)
                ctx = lax.dot_general(
                    w.astype(jnp.bfloat16), vt_bh, (((1,), (1,)), ((), ())),
                    preferred_element_type=jnp.float32)
                ctxs.append((ctx / denom).astype(jnp.bfloat16))
            ctx_all = jnp.concatenate(ctxs, axis=1)
            return jnp.dot(ctx_all, wo_loc,
                           preferred_element_type=jnp.float32)

        acc = [None] * NC
        r0 = [None] * NC
        for b in range(B):
            acc_b = attn_partial(b)
            for ch in range(NC_PER_B):
                c = NC_PER_B * b + ch
                acc[c] = acc_b[:, ch * CW:(ch + 1) * CW]
                send_buf[0, c] = acc[c].astype(jnp.bfloat16)
            if b == 0:
                pl.semaphore_wait(barrier_sem, 2)
            for ch in range(NC_PER_B):
                c = NC_PER_B * b + ch
                r0[c] = exchange(0, c)
                r0[c].start()

        acc2 = [None] * NC
        r1 = [None] * NC
        for c in range(NC):
            r0[c].wait_recv()
            acc2[c] = acc[c] + recv_buf[0, c].astype(jnp.float32)
            send_buf[1, c] = acc2[c].astype(jnp.bfloat16)
            r1[c] = exchange(1, c)
            r1[c].start()

        for c in range(NC):
            r1[c].wait_recv()
            b, ch = c // NC_PER_B, c % NC_PER_B
            out_ref[b, :, ch * CW:(ch + 1) * CW] = (
                acc2[c] + recv_buf[1, c].astype(jnp.float32)
            ).astype(jnp.bfloat16)

        for c in range(NC):
            r0[c].wait_send()
            r1[c].wait_send()

    return pl.pallas_call(
        body,
        out_shape=jax.ShapeDtypeStruct((B, SQ, D_MODEL), jnp.bfloat16),
        in_specs=[pl.BlockSpec(memory_space=pltpu.VMEM)] * 5,
        out_specs=pl.BlockSpec(memory_space=pltpu.VMEM),
        scratch_shapes=[
            pltpu.VMEM((2, NC, SQ, CW), jnp.bfloat16),
            pltpu.VMEM((2, NC, SQ, CW), jnp.bfloat16),
            pltpu.SemaphoreType.DMA((2, NC)),
            pltpu.SemaphoreType.DMA((2, NC)),
        ],
        compiler_params=pltpu.CompilerParams(collective_id=0),
    )(x, wq_loc_out, Kt, Vt, wo_loc_out)


# device time: 13610 ns/iter; 2.5787x vs baseline; 1.0884x over previous
import jax
import jax.numpy as jnp
from jax import lax
from jax.experimental import pallas as pl
from jax.experimental.pallas import tpu as pltpu

N_DEV = 4
B, SQ, SKV, DH = 2, 256, 256, 64
HQ_LOCAL = 4
D_MODEL = 512
D_QKV = HQ_LOCAL * DH
BLK = 64


def kernel(x, Wq, K_ext, V_ext, Wo):
    Kt = jnp.transpose(K_ext, (0, 2, 3, 1))
    Vt = jnp.transpose(V_ext, (0, 2, 3, 1))
    my_pos_out = lax.axis_index("i")
    wq_loc_out = lax.dynamic_slice(Wq, (0, my_pos_out * D_QKV),
                                   (D_MODEL, D_QKV))
    wo_bf16 = Wo.astype(jnp.bfloat16)

    def body(x_ref, wq_ref, k_ref, v_ref, wo_ref, out_ref,
             myctx, nbrrecv, diagrecv,
             prim_send_sems, nbr_recv_sems, fwd_send_sems, diag_recv_sems):
        my_pos = lax.axis_index("i")
        left = (my_pos + N_DEV - 1) % N_DEV
        right = (my_pos + 1) % N_DEV
        diag = jnp.bitwise_xor(my_pos, 2)

        barrier_sem = pltpu.get_barrier_semaphore()
        for nbr in [left, right]:
            pl.semaphore_signal(
                barrier_sem, inc=1,
                device_id=(nbr,), device_id_type=pl.DeviceIdType.MESH,
            )

        HW = D_QKV // 2

        def prim(side, b, half):
            return pltpu.make_async_remote_copy(
                src_ref=myctx.at[b, :, pl.ds(half * HW, HW)],
                dst_ref=nbrrecv.at[side, b, :, pl.ds(half * HW, HW)],
                send_sem=prim_send_sems.at[side, b, half],
                recv_sem=nbr_recv_sems.at[side, b, half],
                device_id=(right if side == 0 else left,),
                device_id_type=pl.DeviceIdType.MESH,
            )

        def fwd(b, half):
            return pltpu.make_async_remote_copy(
                src_ref=nbrrecv.at[b, b, :, pl.ds(half * HW, HW)],
                dst_ref=diagrecv.at[b, :, pl.ds(half * HW, HW)],
                send_sem=fwd_send_sems.at[b, half],
                recv_sem=diag_recv_sems.at[b, half],
                device_id=(right if b == 0 else left,),
                device_id_type=pl.DeviceIdType.MESH,
            )

        wq_loc = wq_ref[...].astype(jnp.bfloat16)

        qb = lax.broadcasted_iota(jnp.int32, (SQ, SKV), 0) // BLK
        kb = lax.broadcasted_iota(jnp.int32, (SQ, SKV), 1) // BLK
        bias = jnp.where(qb == kb, 0.0, -30.0).astype(jnp.float32)

        def head_ctx(q_b, b, h):
            q_bh = q_b[:, h * DH:(h + 1) * DH]
            kt_bh = k_ref[b, h].astype(jnp.bfloat16)
            vt_bh = v_ref[b, h].astype(jnp.bfloat16)
            s = jnp.dot(q_bh, kt_bh,
                        preferred_element_type=jnp.float32) * 0.125 + bias
            w = jnp.exp(s)
            denom = jnp.sum(w, axis=1, keepdims=True)
            ctx = lax.dot_general(
                w.astype(jnp.bfloat16), vt_bh, (((1,), (1,)), ((), ())),
                preferred_element_type=jnp.float32)
            return (ctx / denom).astype(jnp.bfloat16)

        def wo_blk(dev):
            return wo_ref[pl.ds(dev * D_QKV, D_QKV), :]

        def matmul(ctx_bf16, dev):
            return jnp.dot(ctx_bf16, wo_blk(dev),
                           preferred_element_type=jnp.float32)

        ctx = [None, None]
        prims = {}
        for b in range(B):
            q_b = jnp.dot(x_ref[b].astype(jnp.bfloat16), wq_loc,
                          preferred_element_type=jnp.float32
                          ).astype(jnp.bfloat16)
            halves = []
            for half in range(2):
                ctx_h = jnp.concatenate(
                    [head_ctx(q_b, b, 2 * half),
                     head_ctx(q_b, b, 2 * half + 1)], axis=1)
                halves.append(ctx_h)
                myctx[b, :, half * HW:(half + 1) * HW] = ctx_h
                if b == 0 and half == 0:
                    pl.semaphore_wait(barrier_sem, 2)
                for side in range(2):
                    prims[(side, b, half)] = prim(side, b, half)
                    prims[(side, b, half)].start()
            ctx[b] = jnp.concatenate(halves, axis=1)

        acc = [matmul(ctx[b], my_pos) for b in range(B)]

        fwds = {}
        for b in range(B):
            src_side = b
            for half in range(2):
                prims[(src_side, b, half)].wait_recv()
                fwds[(b, half)] = fwd(b, half)
                fwds[(b, half)].start()
            src_dev = left if b == 0 else right
            acc[b] = acc[b] + matmul(nbrrecv[src_side, b], src_dev)

        for b in range(B):
            other_side = 1 - b
            for half in range(2):
                prims[(other_side, b, half)].wait_recv()
            other_dev = right if b == 0 else left
            acc[b] = acc[b] + matmul(nbrrecv[other_side, b], other_dev)

        for b in range(B):
            for half in range(2):
                fwds[(b, half)].wait_recv()
            acc[b] = acc[b] + matmul(diagrecv[b], diag)
            out_ref[b] = acc[b].astype(jnp.bfloat16)

        for key in prims:
            prims[key].wait_send()
        for key in fwds:
            fwds[key].wait_send()

    return pl.pallas_call(
        body,
        out_shape=jax.ShapeDtypeStruct((B, SQ, D_MODEL), jnp.bfloat16),
        in_specs=[pl.BlockSpec(memory_space=pltpu.VMEM)] * 5,
        out_specs=pl.BlockSpec(memory_space=pltpu.VMEM),
        scratch_shapes=[
            pltpu.VMEM((B, SQ, D_QKV), jnp.bfloat16),
            pltpu.VMEM((2, B, SQ, D_QKV), jnp.bfloat16),
            pltpu.VMEM((B, SQ, D_QKV), jnp.bfloat16),
            pltpu.SemaphoreType.DMA((2, B, 2)),
            pltpu.SemaphoreType.DMA((2, B, 2)),
            pltpu.SemaphoreType.DMA((B, 2)),
            pltpu.SemaphoreType.DMA((B, 2)),
        ],
        compiler_params=pltpu.CompilerParams(collective_id=0),
    )(x, wq_loc_out, Kt, Vt, wo_bf16)
